# baseline (speedup 1.0000x reference)
import functools

import jax
import jax.numpy as jnp
import numpy as np
from jax.experimental import pallas as pl
from jax.experimental.pallas import tpu as pltpu

B, S, H, NH, NSTEPS = 512, 101, 128, 8, 8
DK = H // NH
S2 = 128
BB = 128
F32 = jnp.float32
BF16 = jnp.bfloat16


def _decode_body(enc_ref, dem_ref, pool_ref, cap_ref, t_ref, depot_ref,
                 fcwT_ref, fc1_ref, wq_ref, wk_ref, wv_ref, wo_ref,
                 act_ref, logp_ref, depot_out_ref):
    enc = enc_ref[...]
    dem = dem_ref[...]
    pool = pool_ref[...]
    cap0 = cap_ref[0, 0]
    t_val = t_ref[0, 0]
    depot = depot_ref[...]

    e2 = enc.reshape(BB * S2, H)
    k3 = jnp.dot(e2, wk_ref[...],
                 preferred_element_type=F32).reshape(BB, S2, H)
    v3 = jnp.dot(e2, wv_ref[...],
                 preferred_element_type=F32).reshape(BB, S2, H)
    k_t = jnp.swapaxes(k3, 1, 2)
    enc_bf = enc.astype(BF16).astype(F32)
    enc_t_bf = jnp.swapaxes(enc_bf, 1, 2)

    lane_s = jax.lax.broadcasted_iota(jnp.int32, (BB, S2), 1)
    lane_n = jax.lax.broadcasted_iota(jnp.int32, (BB, NSTEPS), 1)
    m_dh = (jax.lax.broadcasted_iota(jnp.int32, (NH, H), 1) // DK
            == jax.lax.broadcasted_iota(jnp.int32, (NH, H), 0)).astype(F32)
    HI = jax.lax.Precision.HIGHEST

    dyn = jnp.full((BB, 1), cap0, F32)
    mask1 = jnp.zeros((BB, S2), F32)
    idx = jnp.zeros((BB, 1), jnp.int32)
    logp_acc = jnp.zeros((BB, 1), F32)
    acts = jnp.zeros((BB, NSTEPS), jnp.int32)
    inp = enc_bf[:, 0, :]

    def update_mask(mask1, dyn, idx):
        upd = ((lane_s == idx) & (idx != 0)).astype(F32)
        mask1 = jnp.maximum(mask1, upd)
        mask = jnp.maximum(mask1, (dem > dyn).astype(F32))
        mask = jnp.where(lane_s == 0, (idx == 0).astype(F32), mask)
        blocked = jnp.where(lane_s == 0, 1.0, (mask > 0).astype(F32))
        allb = jnp.min(blocked, axis=1, keepdims=True)
        mask = jnp.where((lane_s == 0) & (allb > 0), 0.0, mask)
        return mask, mask1

    mask, mask1 = update_mask(mask1, dyn, idx)

    for i in range(NSTEPS):
        cat = jnp.concatenate([inp, dyn], axis=1)
        dec = jnp.dot(cat, fcwT_ref[...], preferred_element_type=F32)
        pool = jnp.dot(pool, fc1_ref[...], preferred_element_type=F32)
        dec = dec + pool
        q = jnp.dot(dec, wq_ref[...], preferred_element_type=F32)

        qm = q[:, None, :] * m_dh[None, :, :]
        sc = jax.lax.dot_general(qm, k_t, (((2,), (1,)), ((0,), (0,))),
                                 precision=HI, preferred_element_type=F32)
        sc = sc / np.sqrt(DK)
        sc = sc - 1e9 * mask[:, None, :]
        sc = sc - jnp.max(sc, axis=2, keepdims=True)
        esc = jnp.exp(sc)
        attn = esc / jnp.sum(esc, axis=2, keepdims=True)

        g3 = jax.lax.dot_general(attn, v3, (((2,), (1,)), ((0,), (0,))),
                                 precision=HI, preferred_element_type=F32)
        glimpse = jnp.sum(g3 * m_dh[:, :], axis=1)
        glimpse = glimpse.astype(BF16).astype(F32)
        glimpse = jnp.dot(glimpse, wo_ref[...], preferred_element_type=F32)

        raw = jax.lax.dot_general(glimpse[:, None, :], enc_t_bf,
                                  (((2,), (1,)), ((0,), (0,))),
                                  preferred_element_type=F32)[:, 0, :]
        logits = raw / np.sqrt(H)
        logits = 10.0 * jnp.tanh(logits)
        logits = jnp.where(mask > 0, -1e9, logits) / t_val
        logits = logits - jnp.max(logits, axis=1, keepdims=True)
        el = jnp.exp(logits)
        p = el / jnp.sum(el, axis=1, keepdims=True)

        pmax = jnp.max(p, axis=1, keepdims=True)
        idx = jnp.min(jnp.where(p == pmax, lane_s, S2), axis=1, keepdims=True)
        acts = jnp.where(lane_n == i, idx, acts)
        onehot = (lane_s == idx).astype(F32)
        psel = jnp.sum(p * onehot, axis=1, keepdims=True)
        isdone = (jnp.sum(jnp.where(lane_s >= 1, mask1, 0.0), axis=1,
                          keepdims=True) >= (S - 1)).astype(F32)
        logp_acc = logp_acc + jnp.log(psel + 1e-12) * (1.0 - isdone)

        sel = jnp.sum(dem * onehot, axis=1, keepdims=True)
        godep = (idx == 0)
        dyn = jnp.where(godep, cap0, dyn - sel)
        depot = depot + godep.astype(F32)
        mask, mask1 = update_mask(mask1, dyn, idx)
        inp = jax.lax.dot_general(onehot[:, None, :], enc_bf,
                                  (((2,), (1,)), ((0,), (0,))),
                                  preferred_element_type=F32)[:, 0, :]

    act_ref[...] = acts
    logp_ref[...] = logp_acc
    depot_out_ref[...] = depot


@functools.partial(jax.jit, static_argnames=("interpret",))
def _run(encp, demp, pool, cap, t_arr, depot, fcwT, fc1t, wqt, wkt, wvt,
         wot, interpret=False):
    grid = (B // BB,)
    full = lambda shp: pl.BlockSpec(shp, lambda i: (0,) * len(shp))
    blk = lambda shp: pl.BlockSpec(shp, lambda i: (i,) + (0,) * (len(shp) - 1))
    return pl.pallas_call(
        _decode_body,
        grid=grid,
        in_specs=[
            blk((BB, S2, H)), blk((BB, S2)), blk((BB, H)),
            full((1, 1)), full((1, 1)), blk((BB, 1)),
            full((H + 1, H)), full((H, H)),
            full((H, H)), full((H, H)), full((H, H)), full((H, H)),
        ],
        out_specs=[blk((BB, NSTEPS)), blk((BB, 1)), blk((BB, 1))],
        out_shape=[
            jax.ShapeDtypeStruct((B, NSTEPS), jnp.int32),
            jax.ShapeDtypeStruct((B, 1), F32),
            jax.ShapeDtypeStruct((B, 1), F32),
        ],
        compiler_params=pltpu.CompilerParams(
            dimension_semantics=("parallel",)),
        interpret=interpret,
    )(encp, demp, pool, cap, t_arr, depot, fcwT, fc1t, wqt, wkt, wvt, wot)


def kernel(encoder_inputs, pool, capacity, demand, n_steps, T, greedy,
           depot_visits, fc_w, fc1_w, Wq, Wk, Wv, Wo, *, interpret=False):
    del n_steps, greedy
    encp = jnp.pad(encoder_inputs, ((0, 0), (0, S2 - S), (0, 0)))
    demp = jnp.pad(demand, ((0, 0), (0, S2 - S)), constant_values=2.0)
    cap = capacity.astype(F32)
    t_arr = jnp.asarray(T, F32).reshape(1, 1)
    depot = depot_visits.astype(F32)[:, None]
    acts, logp, depot_out = _run(
        encp, demp, pool, cap, t_arr, depot,
        fc_w.T, fc1_w.T, Wq.T, Wk.T, Wv.T, Wo.T, interpret=interpret)
    return acts, logp[:, 0], depot_out[:, 0]


# speedup vs baseline: 2.0033x; 2.0033x over previous
import functools

import jax
import jax.numpy as jnp
import numpy as np
from jax.experimental import pallas as pl
from jax.experimental.pallas import tpu as pltpu

B, S, H, NH, NSTEPS = 512, 101, 128, 8, 8
DK = H // NH
S2 = 128
BB = 128
F32 = jnp.float32
BF16 = jnp.bfloat16


def _decode_body(enc_ref, dem_ref, pool_ref, cap_ref, t_ref, depot_ref,
                 fcwT_ref, fc1_ref, wq_ref, wk_ref, wv_ref, wo_ref,
                 act_ref, logp_ref, depot_out_ref):
    enc = enc_ref[...]
    dem = dem_ref[...]
    pool = pool_ref[...]
    cap0 = cap_ref[0, 0]
    t_val = t_ref[0, 0]
    depot = depot_ref[...]

    e2 = enc.reshape(BB * S2, H)
    k3 = jnp.dot(e2, wk_ref[...],
                 preferred_element_type=F32).reshape(BB, S2, H)
    v3 = jnp.dot(e2, wv_ref[...],
                 preferred_element_type=F32).reshape(BB, S2, H)
    k_t = jnp.swapaxes(k3, 1, 2)
    v_t = jnp.swapaxes(v3, 1, 2)
    enc_bf = enc.astype(BF16).astype(F32)
    enc_t_bf = jnp.swapaxes(enc_bf, 1, 2)

    lane_s = jax.lax.broadcasted_iota(jnp.int32, (BB, S2), 1)
    lane_n = jax.lax.broadcasted_iota(jnp.int32, (BB, NSTEPS), 1)

    dyn = jnp.full((BB, 1), cap0, F32)
    mask1 = jnp.zeros((BB, S2), F32)
    idx = jnp.zeros((BB, 1), jnp.int32)
    logp_acc = jnp.zeros((BB, 1), F32)
    acts = jnp.zeros((BB, NSTEPS), jnp.int32)
    inp = enc_bf[:, 0, :]

    def update_mask(mask1, dyn, idx):
        upd = ((lane_s == idx) & (idx != 0)).astype(F32)
        mask1 = jnp.maximum(mask1, upd)
        mask = jnp.maximum(mask1, (dem > dyn).astype(F32))
        mask = jnp.where(lane_s == 0, (idx == 0).astype(F32), mask)
        blocked = jnp.where(lane_s == 0, 1.0, (mask > 0).astype(F32))
        allb = jnp.min(blocked, axis=1, keepdims=True)
        mask = jnp.where((lane_s == 0) & (allb > 0), 0.0, mask)
        return mask, mask1

    mask, mask1 = update_mask(mask1, dyn, idx)

    for i in range(NSTEPS):
        cat = jnp.concatenate([inp, dyn], axis=1)
        dec = jnp.dot(cat, fcwT_ref[...], preferred_element_type=F32)
        pool = jnp.dot(pool, fc1_ref[...], preferred_element_type=F32)
        dec = dec + pool
        q = jnp.dot(dec, wq_ref[...], preferred_element_type=F32)

        prod = k_t * q[:, :, None]
        sc = jnp.sum(prod.reshape(BB, NH, DK, S2), axis=2)
        sc = sc / np.sqrt(DK)
        sc = sc - 1e9 * mask[:, None, :]
        sc = sc - jnp.max(sc, axis=2, keepdims=True)
        esc = jnp.exp(sc)
        attn = esc / jnp.sum(esc, axis=2, keepdims=True)

        aexp = jnp.broadcast_to(attn[:, :, None, :],
                                (BB, NH, DK, S2)).reshape(BB, H, S2)
        glimpse = jnp.sum(aexp * v_t, axis=2)
        glimpse = glimpse.astype(BF16).astype(F32)
        glimpse = jnp.dot(glimpse, wo_ref[...], preferred_element_type=F32)

        raw = jax.lax.dot_general(glimpse[:, None, :], enc_t_bf,
                                  (((2,), (1,)), ((0,), (0,))),
                                  preferred_element_type=F32)[:, 0, :]
        logits = raw / np.sqrt(H)
        logits = 10.0 * jnp.tanh(logits)
        logits = jnp.where(mask > 0, -1e9, logits) / t_val
        logits = logits - jnp.max(logits, axis=1, keepdims=True)
        el = jnp.exp(logits)
        p = el / jnp.sum(el, axis=1, keepdims=True)

        pmax = jnp.max(p, axis=1, keepdims=True)
        idx = jnp.min(jnp.where(p == pmax, lane_s, S2), axis=1, keepdims=True)
        acts = jnp.where(lane_n == i, idx, acts)
        onehot = (lane_s == idx).astype(F32)
        psel = jnp.sum(p * onehot, axis=1, keepdims=True)
        isdone = (jnp.sum(jnp.where(lane_s >= 1, mask1, 0.0), axis=1,
                          keepdims=True) >= (S - 1)).astype(F32)
        logp_acc = logp_acc + jnp.log(psel + 1e-12) * (1.0 - isdone)

        sel = jnp.sum(dem * onehot, axis=1, keepdims=True)
        godep = (idx == 0)
        dyn = jnp.where(godep, cap0, dyn - sel)
        depot = depot + godep.astype(F32)
        mask, mask1 = update_mask(mask1, dyn, idx)
        inp = jax.lax.dot_general(onehot[:, None, :], enc_bf,
                                  (((2,), (1,)), ((0,), (0,))),
                                  preferred_element_type=F32)[:, 0, :]

    act_ref[...] = acts
    logp_ref[...] = logp_acc
    depot_out_ref[...] = depot


@functools.partial(jax.jit, static_argnames=("interpret",))
def _run(encp, demp, pool, cap, t_arr, depot, fcwT, fc1t, wqt, wkt, wvt,
         wot, interpret=False):
    grid = (B // BB,)
    full = lambda shp: pl.BlockSpec(shp, lambda i: (0,) * len(shp))
    blk = lambda shp: pl.BlockSpec(shp, lambda i: (i,) + (0,) * (len(shp) - 1))
    return pl.pallas_call(
        _decode_body,
        grid=grid,
        in_specs=[
            blk((BB, S2, H)), blk((BB, S2)), blk((BB, H)),
            full((1, 1)), full((1, 1)), blk((BB, 1)),
            full((H + 1, H)), full((H, H)),
            full((H, H)), full((H, H)), full((H, H)), full((H, H)),
        ],
        out_specs=[blk((BB, NSTEPS)), blk((BB, 1)), blk((BB, 1))],
        out_shape=[
            jax.ShapeDtypeStruct((B, NSTEPS), jnp.int32),
            jax.ShapeDtypeStruct((B, 1), F32),
            jax.ShapeDtypeStruct((B, 1), F32),
        ],
        compiler_params=pltpu.CompilerParams(
            dimension_semantics=("parallel",)),
        interpret=interpret,
    )(encp, demp, pool, cap, t_arr, depot, fcwT, fc1t, wqt, wkt, wvt, wot)


def kernel(encoder_inputs, pool, capacity, demand, n_steps, T, greedy,
           depot_visits, fc_w, fc1_w, Wq, Wk, Wv, Wo, *, interpret=False):
    del n_steps, greedy
    encp = jnp.pad(encoder_inputs, ((0, 0), (0, S2 - S), (0, 0)))
    demp = jnp.pad(demand, ((0, 0), (0, S2 - S)), constant_values=2.0)
    cap = capacity.astype(F32)
    t_arr = jnp.asarray(T, F32).reshape(1, 1)
    depot = depot_visits.astype(F32)[:, None]
    acts, logp, depot_out = _run(
        encp, demp, pool, cap, t_arr, depot,
        fc_w.T, fc1_w.T, Wq.T, Wk.T, Wv.T, Wo.T, interpret=interpret)
    return acts, logp[:, 0], depot_out[:, 0]
